# baseline (device time: 104073 ns/iter reference)
import jax
import jax.numpy as jnp
from jax import lax
from jax.experimental import pallas as pl
from jax.experimental.pallas import tpu as pltpu

N_DEV = 4


def kernel(x, Win0, Wout0, Win1, Wout1, Win2, Wout2):
    b, d = x.shape
    h_sh = Win0.shape[1]
    H = h_sh * N_DEV

    def body(x_ref, wi0_ref, wo0_ref, wi1_ref, wo1_ref, wi2_ref, wo2_ref,
             out_ref,
             wi_stage, wo_stage, wi_comm, wo_comm, wi_full, wo_full,
             y_stage, y_comm,
             wi_send, wi_recv, wo_send, wo_recv, y_send, y_recv):
        my = lax.axis_index("i")
        left = lax.rem(my + N_DEV - 1, N_DEV)
        right = lax.rem(my + 1, N_DEV)

        barrier = pltpu.get_barrier_semaphore()
        for nbr in (left, right):
            pl.semaphore_signal(barrier, inc=1, device_id=(nbr,),
                                device_id_type=pl.DeviceIdType.MESH)
        pl.semaphore_wait(barrier, 2)

        wi_refs = [wi0_ref, wi1_ref, wi2_ref]
        wo_refs = [wo0_ref, wo1_ref, wo2_ref]

        for l in range(3):
            wi_stage[l, :, :] = wi_refs[l][...].astype(jnp.bfloat16)
            wo_stage[l, :, :] = wo_refs[l][...].astype(jnp.bfloat16)
            wi_full[l, :, pl.ds(my * h_sh, h_sh)] = wi_stage[l]
            wo_full[l, pl.ds(my * h_sh, h_sh), :] = wo_stage[l]

        for l in range(3):
            for h in range(N_DEV - 1):
                origin = lax.rem(my - h - 1 + N_DEV, N_DEV)

                wi_src = wi_stage.at[l] if h == 0 else wi_comm.at[l, h - 1]
                rdma_wi = pltpu.make_async_remote_copy(
                    src_ref=wi_src,
                    dst_ref=wi_comm.at[l, h],
                    send_sem=wi_send.at[l, h],
                    recv_sem=wi_recv.at[l, h],
                    device_id=(right,),
                    device_id_type=pl.DeviceIdType.MESH,
                )
                rdma_wi.start()

                wo_src = wo_stage.at[l] if h == 0 else wo_comm.at[l, h - 1]
                rdma_wo = pltpu.make_async_remote_copy(
                    src_ref=wo_src,
                    dst_ref=wo_comm.at[l, h],
                    send_sem=wo_send.at[l, h],
                    recv_sem=wo_recv.at[l, h],
                    device_id=(right,),
                    device_id_type=pl.DeviceIdType.MESH,
                )
                rdma_wo.start()

                rdma_wi.wait()
                rdma_wo.wait()
                wi_full[l, :, pl.ds(origin * h_sh, h_sh)] = wi_comm[l, h]
                wo_full[l, pl.ds(origin * h_sh, h_sh), :] = wo_comm[l, h]

        acts = x_ref[...].astype(jnp.bfloat16)
        y = None
        for l in range(3):
            hmat = jnp.dot(acts, wi_full[l],
                           preferred_element_type=jnp.float32)
            hmat = jnp.maximum(hmat, 0.0).astype(jnp.bfloat16)
            y = jnp.dot(hmat, wo_full[l],
                        preferred_element_type=jnp.float32)
            acts = y.astype(jnp.bfloat16)

        y_stage[...] = y
        out_ref[pl.ds(my * b, b), :] = y

        for h in range(N_DEV - 1):
            src = y_stage if h == 0 else y_comm.at[h - 1]
            rdma = pltpu.make_async_remote_copy(
                src_ref=src,
                dst_ref=y_comm.at[h],
                send_sem=y_send.at[h],
                recv_sem=y_recv.at[h],
                device_id=(right,),
                device_id_type=pl.DeviceIdType.MESH,
            )
            rdma.start()
            rdma.wait()
            origin = lax.rem(my - h - 1 + N_DEV, N_DEV)
            out_ref[pl.ds(origin * b, b), :] = y_comm[h]

    return pl.pallas_call(
        body,
        out_shape=jax.ShapeDtypeStruct((N_DEV * b, d), jnp.float32),
        in_specs=[pl.BlockSpec(memory_space=pltpu.VMEM)] * 7,
        out_specs=pl.BlockSpec(memory_space=pltpu.VMEM),
        scratch_shapes=[
            pltpu.VMEM((3, d, h_sh), jnp.bfloat16),
            pltpu.VMEM((3, h_sh, d), jnp.bfloat16),
            pltpu.VMEM((3, N_DEV - 1, d, h_sh), jnp.bfloat16),
            pltpu.VMEM((3, N_DEV - 1, h_sh, d), jnp.bfloat16),
            pltpu.VMEM((3, d, H), jnp.bfloat16),
            pltpu.VMEM((3, H, d), jnp.bfloat16),
            pltpu.VMEM((b, d), jnp.float32),
            pltpu.VMEM((N_DEV - 1, b, d), jnp.float32),
            pltpu.SemaphoreType.DMA((3, N_DEV - 1)),
            pltpu.SemaphoreType.DMA((3, N_DEV - 1)),
            pltpu.SemaphoreType.DMA((3, N_DEV - 1)),
            pltpu.SemaphoreType.DMA((3, N_DEV - 1)),
            pltpu.SemaphoreType.DMA((N_DEV - 1,)),
            pltpu.SemaphoreType.DMA((N_DEV - 1,)),
        ],
        compiler_params=pltpu.CompilerParams(collective_id=0),
    )(x, Win0, Wout0, Win1, Wout1, Win2, Wout2)


# device time: 59789 ns/iter; 1.7407x vs baseline; 1.7407x over previous
import jax
import jax.numpy as jnp
from jax import lax
from jax.experimental import pallas as pl
from jax.experimental.pallas import tpu as pltpu

N_DEV = 4


def kernel(x, Win0, Wout0, Win1, Wout1, Win2, Wout2):
    b, d = x.shape
    h_sh = Win0.shape[1]
    bp = 2 * b

    def body(x_ref, wi0_ref, wo0_ref, wi1_ref, wo1_ref, wi2_ref, wo2_ref,
             out_ref,
             x_stage, x_recv, x_pair,
             wi_mine, wo_mine, wi_cross, wo_cross,
             p_send, p_recv, y_recv,
             w_send, w_recv, x_send_sem, x_recv_sem,
             p_send_sems, p_recv_sems, y_send_sem, y_recv_sem):
        my = lax.axis_index("i")
        myslot = lax.rem(my, 2)
        pair = my + 1 - 2 * myslot
        cross = N_DEV - 1 - my
        base = (my // 2) * bp
        other = bp - base

        barrier = pltpu.get_barrier_semaphore()
        for nbr in (pair, cross):
            pl.semaphore_signal(barrier, inc=1, device_id=(nbr,),
                                device_id_type=pl.DeviceIdType.MESH)
        pl.semaphore_wait(barrier, 2)

        x_stage[...] = x_ref[...].astype(jnp.bfloat16)
        wi_refs = [wi0_ref, wi1_ref, wi2_ref]
        wo_refs = [wo0_ref, wo1_ref, wo2_ref]
        for l in range(3):
            wi_mine[l, :, :] = wi_refs[l][...].astype(jnp.bfloat16)
            wo_mine[l, :, :] = wo_refs[l][...].astype(jnp.bfloat16)

        x_rdma = pltpu.make_async_remote_copy(
            src_ref=x_stage, dst_ref=x_recv,
            send_sem=x_send_sem.at[0], recv_sem=x_recv_sem.at[0],
            device_id=(pair,), device_id_type=pl.DeviceIdType.MESH,
        )
        x_rdma.start()
        x_rdma.wait()

        w_rdmas = []
        for l in range(3):
            r_wi = pltpu.make_async_remote_copy(
                src_ref=wi_mine.at[l], dst_ref=wi_cross.at[l],
                send_sem=w_send.at[l], recv_sem=w_recv.at[l],
                device_id=(cross,), device_id_type=pl.DeviceIdType.MESH,
            )
            r_wi.start()
            r_wo = pltpu.make_async_remote_copy(
                src_ref=wo_mine.at[l], dst_ref=wo_cross.at[l],
                send_sem=w_send.at[3 + l], recv_sem=w_recv.at[3 + l],
                device_id=(cross,), device_id_type=pl.DeviceIdType.MESH,
            )
            r_wo.start()
            w_rdmas.append((r_wi, r_wo))

        x_pair[pl.ds(myslot * b, b), :] = x_stage[...]
        x_pair[pl.ds((1 - myslot) * b, b), :] = x_recv[...]

        acts = x_pair[...]
        xn = None
        for l in range(3):
            r_wi, r_wo = w_rdmas[l]
            r_wi.wait()
            r_wo.wait()
            h1 = jnp.dot(acts, wi_mine[l], preferred_element_type=jnp.float32)
            h2 = jnp.dot(acts, wi_cross[l], preferred_element_type=jnp.float32)
            h1 = jnp.maximum(h1, 0.0).astype(jnp.bfloat16)
            h2 = jnp.maximum(h2, 0.0).astype(jnp.bfloat16)
            partial = (
                jnp.dot(h1, wo_mine[l], preferred_element_type=jnp.float32)
                + jnp.dot(h2, wo_cross[l], preferred_element_type=jnp.float32)
            )
            p_send[l, :, :] = partial.astype(jnp.bfloat16)
            p_rdma = pltpu.make_async_remote_copy(
                src_ref=p_send.at[l], dst_ref=p_recv.at[l],
                send_sem=p_send_sems.at[l], recv_sem=p_recv_sems.at[l],
                device_id=(pair,), device_id_type=pl.DeviceIdType.MESH,
            )
            p_rdma.start()
            p_rdma.wait()
            xn = p_send[l].astype(jnp.float32) + p_recv[l].astype(jnp.float32)
            acts = xn.astype(jnp.bfloat16)

        out_ref[pl.ds(base, bp), :] = xn

        x_pair[...] = acts
        y_rdma = pltpu.make_async_remote_copy(
            src_ref=x_pair, dst_ref=y_recv,
            send_sem=y_send_sem.at[0], recv_sem=y_recv_sem.at[0],
            device_id=(cross,), device_id_type=pl.DeviceIdType.MESH,
        )
        y_rdma.start()
        y_rdma.wait()
        out_ref[pl.ds(other, bp), :] = y_recv[...].astype(jnp.float32)

    return pl.pallas_call(
        body,
        out_shape=jax.ShapeDtypeStruct((N_DEV * b, d), jnp.float32),
        in_specs=[pl.BlockSpec(memory_space=pltpu.VMEM)] * 7,
        out_specs=pl.BlockSpec(memory_space=pltpu.VMEM),
        scratch_shapes=[
            pltpu.VMEM((b, d), jnp.bfloat16),
            pltpu.VMEM((b, d), jnp.bfloat16),
            pltpu.VMEM((bp, d), jnp.bfloat16),
            pltpu.VMEM((3, d, h_sh), jnp.bfloat16),
            pltpu.VMEM((3, h_sh, d), jnp.bfloat16),
            pltpu.VMEM((3, d, h_sh), jnp.bfloat16),
            pltpu.VMEM((3, h_sh, d), jnp.bfloat16),
            pltpu.VMEM((3, bp, d), jnp.bfloat16),
            pltpu.VMEM((3, bp, d), jnp.bfloat16),
            pltpu.VMEM((bp, d), jnp.bfloat16),
            pltpu.SemaphoreType.DMA((6,)),
            pltpu.SemaphoreType.DMA((6,)),
            pltpu.SemaphoreType.DMA((1,)),
            pltpu.SemaphoreType.DMA((1,)),
            pltpu.SemaphoreType.DMA((3,)),
            pltpu.SemaphoreType.DMA((3,)),
            pltpu.SemaphoreType.DMA((1,)),
            pltpu.SemaphoreType.DMA((1,)),
        ],
        compiler_params=pltpu.CompilerParams(collective_id=0),
    )(x, Win0, Wout0, Win1, Wout1, Win2, Wout2)


# device time: 43956 ns/iter; 2.3677x vs baseline; 1.3602x over previous
import jax
import jax.numpy as jnp
from jax import lax
from jax.experimental import pallas as pl
from jax.experimental.pallas import tpu as pltpu

N_DEV = 4


def kernel(x, Win0, Wout0, Win1, Wout1, Win2, Wout2):
    b, d = x.shape
    h_sh = Win0.shape[1]
    bp = 2 * b

    def body(x_ref, wi0_ref, wo0_ref, wi1_ref, wo1_ref, wi2_ref, wo2_ref,
             out_ref,
             x_stage, x_recv, x_pair,
             wi_mine, wo_mine, wi_cross, wo_cross,
             p_send, p_recv, y_stage, y_recv,
             w_send, w_recv, x_send_sem, x_recv_sem,
             p_send_sems, p_recv_sems, y_send_sems, y_recv_sems):
        my = lax.axis_index("i")
        myslot = lax.rem(my, 2)
        pair = my + 1 - 2 * myslot
        cross = N_DEV - 1 - my
        base = (my // 2) * bp
        other = bp - base

        barrier = pltpu.get_barrier_semaphore()
        for nbr in (pair, cross):
            pl.semaphore_signal(barrier, inc=1, device_id=(nbr,),
                                device_id_type=pl.DeviceIdType.MESH)
        pl.semaphore_wait(barrier, 2)

        wi_refs = [wi0_ref, wi1_ref, wi2_ref]
        wo_refs = [wo0_ref, wo1_ref, wo2_ref]
        w_rdmas = []
        for l in range(3):
            wi_mine[l, :, :] = wi_refs[l][...].astype(jnp.bfloat16)
            wo_mine[l, :, :] = wo_refs[l][...].astype(jnp.bfloat16)
            r_wi = pltpu.make_async_remote_copy(
                src_ref=wi_mine.at[l], dst_ref=wi_cross.at[l],
                send_sem=w_send.at[l], recv_sem=w_recv.at[l],
                device_id=(cross,), device_id_type=pl.DeviceIdType.MESH,
            )
            r_wi.start()
            r_wo = pltpu.make_async_remote_copy(
                src_ref=wo_mine.at[l], dst_ref=wo_cross.at[l],
                send_sem=w_send.at[3 + l], recv_sem=w_recv.at[3 + l],
                device_id=(cross,), device_id_type=pl.DeviceIdType.MESH,
            )
            r_wo.start()
            w_rdmas.append((r_wi, r_wo))

        x_stage[...] = x_ref[...].astype(jnp.bfloat16)
        x_rdma = pltpu.make_async_remote_copy(
            src_ref=x_stage, dst_ref=x_recv,
            send_sem=x_send_sem.at[0], recv_sem=x_recv_sem.at[0],
            device_id=(pair,), device_id_type=pl.DeviceIdType.MESH,
        )
        x_rdma.start()
        x_rdma.wait()

        x_pair[pl.ds(myslot * b, b), :] = x_stage[...]
        x_pair[pl.ds((1 - myslot) * b, b), :] = x_recv[...]

        def compute_send(l, c, a):
            h1 = jnp.dot(a, wi_mine[l], preferred_element_type=jnp.float32)
            h2 = jnp.dot(a, wi_cross[l], preferred_element_type=jnp.float32)
            h1 = jnp.maximum(h1, 0.0).astype(jnp.bfloat16)
            h2 = jnp.maximum(h2, 0.0).astype(jnp.bfloat16)
            p = (
                jnp.dot(h1, wo_mine[l], preferred_element_type=jnp.float32)
                + jnp.dot(h2, wo_cross[l], preferred_element_type=jnp.float32)
            )
            p_send[l, c, :, :] = p.astype(jnp.bfloat16)
            r = pltpu.make_async_remote_copy(
                src_ref=p_send.at[l, c], dst_ref=p_recv.at[l, c],
                send_sem=p_send_sems.at[l, c], recv_sem=p_recv_sems.at[l, c],
                device_id=(pair,), device_id_type=pl.DeviceIdType.MESH,
            )
            r.start()
            return r

        def reduce(l, c, r):
            r.wait()
            return (p_send[l, c].astype(jnp.float32)
                    + p_recv[l, c].astype(jnp.float32))

        rds = {}
        w_rdmas[0][0].wait()
        w_rdmas[0][1].wait()
        rds[(0, 0)] = compute_send(0, 0, x_pair[0:b, :])
        rds[(0, 1)] = compute_send(0, 1, x_pair[b:bp, :])
        for l in (1, 2):
            w_rdmas[l][0].wait()
            w_rdmas[l][1].wait()
            for c in (0, 1):
                xn = reduce(l - 1, c, rds[(l - 1, c)])
                rds[(l, c)] = compute_send(l, c, xn.astype(jnp.bfloat16))

        y_rdmas = []
        for c in (0, 1):
            xn = reduce(2, c, rds[(2, c)])
            out_ref[pl.ds(base + c * b, b), :] = xn
            y_stage[c, :, :] = xn.astype(jnp.bfloat16)
            yr = pltpu.make_async_remote_copy(
                src_ref=y_stage.at[c], dst_ref=y_recv.at[c],
                send_sem=y_send_sems.at[c], recv_sem=y_recv_sems.at[c],
                device_id=(cross,), device_id_type=pl.DeviceIdType.MESH,
            )
            yr.start()
            y_rdmas.append(yr)
        for c in (0, 1):
            y_rdmas[c].wait()
            out_ref[pl.ds(other + c * b, b), :] = y_recv[c].astype(jnp.float32)

    return pl.pallas_call(
        body,
        out_shape=jax.ShapeDtypeStruct((N_DEV * b, d), jnp.float32),
        in_specs=[pl.BlockSpec(memory_space=pltpu.VMEM)] * 7,
        out_specs=pl.BlockSpec(memory_space=pltpu.VMEM),
        scratch_shapes=[
            pltpu.VMEM((b, d), jnp.bfloat16),
            pltpu.VMEM((b, d), jnp.bfloat16),
            pltpu.VMEM((bp, d), jnp.bfloat16),
            pltpu.VMEM((3, d, h_sh), jnp.bfloat16),
            pltpu.VMEM((3, h_sh, d), jnp.bfloat16),
            pltpu.VMEM((3, d, h_sh), jnp.bfloat16),
            pltpu.VMEM((3, h_sh, d), jnp.bfloat16),
            pltpu.VMEM((3, 2, b, d), jnp.bfloat16),
            pltpu.VMEM((3, 2, b, d), jnp.bfloat16),
            pltpu.VMEM((2, b, d), jnp.bfloat16),
            pltpu.VMEM((2, b, d), jnp.bfloat16),
            pltpu.SemaphoreType.DMA((6,)),
            pltpu.SemaphoreType.DMA((6,)),
            pltpu.SemaphoreType.DMA((1,)),
            pltpu.SemaphoreType.DMA((1,)),
            pltpu.SemaphoreType.DMA((3, 2)),
            pltpu.SemaphoreType.DMA((3, 2)),
            pltpu.SemaphoreType.DMA((2,)),
            pltpu.SemaphoreType.DMA((2,)),
        ],
        compiler_params=pltpu.CompilerParams(collective_id=0),
    )(x, Win0, Wout0, Win1, Wout1, Win2, Wout2)
